# baseline (speedup 1.0000x reference)
import functools

import jax
import jax.numpy as jnp
from jax import lax
from jax.experimental import pallas as pl
from jax.experimental.pallas import tpu as pltpu
from jax.experimental.pallas import tpu_sc as plsc

B, S, D = 16, 2048, 64
L = 16
NC, NS = 2, 16
NW = NC * NS
HALF = S // 2
CHUNK = 128
NCHUNK = HALF // CHUNK
WBAND = 80
EPS = 1e-5


def _rsqrt(v):
    i = plsc.bitcast(v, jnp.int32)
    i = jnp.int32(0x5F3759DF) - lax.shift_right_arithmetic(i, 1)
    y = plsc.bitcast(i, jnp.float32)
    half = v * 0.5
    for _ in range(3):
        y = y * (1.5 - half * y * y)
    return y


_mesh = plsc.VectorSubcoreMesh(core_axis_name="c", subcore_axis_name="s")


@functools.partial(
    pl.kernel,
    out_type=jax.ShapeDtypeStruct((B, D, S), jnp.float32),
    mesh=_mesh,
    scratch_types=[
        pltpu.VMEM((HALF,), jnp.int32),
        pltpu.VMEM((HALF,), jnp.int32),
        pltpu.VMEM((HALF,), jnp.int32),
        pltpu.VMEM((HALF,), jnp.int32),
        pltpu.VMEM((HALF,), jnp.int32),
        pltpu.VMEM((3, CHUNK, 128), jnp.float32),
        pltpu.VMEM((2, WBAND, 128), jnp.float32),
        pltpu.VMEM((2, CHUNK // 2, 128), jnp.float32),
        pltpu.VMEM((2, D, CHUNK), jnp.float32),
        pltpu.VMEM((128,), jnp.float32),
        pltpu.VMEM((D,), jnp.float32),
        pltpu.VMEM((D,), jnp.float32),
        pltpu.SemaphoreType.DMA,
        pltpu.SemaphoreType.DMA,
        pltpu.SemaphoreType.DMA,
        pltpu.SemaphoreType.DMA,
        pltpu.SemaphoreType.DMA,
        pltpu.SemaphoreType.DMA,
        pltpu.SemaphoreType.DMA,
        pltpu.SemaphoreType.DMA,
        pltpu.SemaphoreType.DMA,
    ],
    compiler_params=pltpu.CompilerParams(
        needs_layout_passes=False, use_tc_tiling_on_sc=True),
)
def _emb_ln(ids_hbm, ws_hbm, tok_hbm, pos_hbm, wst_hbm, word_hbm,
            gamma_hbm, beta_hbm, out_hbm,
            ids_all, ws_all, prev_half, idx_half, cs_all,
            tok_rows, word_rows, pos_rows, ycols, wst_v, gamma_v, beta_v,
            tsem0, tsem1, tsem2, wsem0, wsem1, psem0, psem1, osem0, osem1):
    cid = lax.axis_index("c")
    sid = lax.axis_index("s")
    wid = sid * NC + cid
    b = wid // 2
    h = wid % 2
    base = pl.multiple_of(h * HALF, HALF)
    row0 = pl.multiple_of(b * S, S)

    tsem = (tsem0, tsem1, tsem2)
    wsem = (wsem0, wsem1)
    psem = (psem0, psem1)
    osem = (osem0, osem1)

    pltpu.sync_copy(gamma_hbm, gamma_v)
    pltpu.sync_copy(beta_hbm, beta_v)
    pltpu.sync_copy(wst_hbm, wst_v)
    my0 = pl.multiple_of(row0 + base, HALF)
    pltpu.sync_copy(ids_hbm.at[pl.ds(my0, HALF)], ids_all)
    pltpu.sync_copy(ws_hbm.at[pl.ds(my0, HALF)], ws_all)
    pltpu.sync_copy(ws_hbm.at[pl.ds(row0, HALF)], prev_half)

    def _sum_body(i, acc):
        return acc + prev_half[pl.ds(i * L, L)]

    acc = lax.fori_loop(0, HALF // L, _sum_body, jnp.zeros((L,), jnp.int32))
    offset0 = jnp.where(h == 1, jnp.sum(acc), 0)

    chunk_off = []
    off = offset0

    def _cs_step(i, o):
        v = ws_all[pl.ds(i * L, L)]
        cs_all[pl.ds(i * L, L)] = lax.cumsum(v, axis=0) + o
        return o + jnp.sum(v)

    for c in range(NCHUNK):
        chunk_off.append(off)
        off = lax.fori_loop(c * (CHUNK // L), (c + 1) * (CHUNK // L),
                            _cs_step, off)

    def _ih_step(i, _):
        idx_half[pl.ds(i * L, L)] = lax.shift_right_logical(
            ids_all[pl.ds(i * L, L)], 1)
        return 0

    lax.fori_loop(0, HALF // L, _ih_step, 0)

    k0 = [pl.multiple_of((chunk_off[c] // 2) & ~jnp.int32(7), 8)
          for c in range(NCHUNK)]

    def issue_tok(c):
        return pltpu.async_copy(
            tok_hbm.at[idx_half.at[pl.ds(c * CHUNK, CHUNK)]],
            tok_rows.at[c % 3], tsem[c % 3])

    def issue_word(c):
        return pltpu.async_copy(
            word_hbm.at[pl.ds(k0[c], WBAND), :],
            word_rows.at[c % 2], wsem[c % 2])

    def issue_pos(c):
        p0 = pl.multiple_of((base + c * CHUNK) // 2, CHUNK // 2)
        return pltpu.async_copy(
            pos_hbm.at[pl.ds(p0, CHUNK // 2), :],
            pos_rows.at[c % 2], psem[c % 2])

    lane = lax.iota(jnp.int32, L)
    tok_pend = {c: issue_tok(c) for c in range(min(3, NCHUNK))}
    word_pend = {c: issue_word(c) for c in range(min(2, NCHUNK))}
    pos_pend = {c: issue_pos(c) for c in range(min(2, NCHUNK))}
    out_pend = [None, None]

    for c in range(NCHUNK):
        s3 = c % 3
        s2 = c % 2
        tok_pend.pop(c).wait()
        word_pend.pop(c).wait()
        pos_pend.pop(c).wait()
        if out_pend[s2] is not None:
            out_pend[s2].wait()

        tok_s = tok_rows.at[s3]
        word_s = word_rows.at[s2]
        pos_s = pos_rows.at[s2]
        y_s = ycols.at[s2]
        k0c = k0[c]

        def group_body(g, _):
            o = c * CHUNK + g * L
            slots = g * L + lane
            ids16 = ids_all[pl.ds(o, L)]
            cs16 = cs_all[pl.ds(o, L)]
            ws16 = ws_all[pl.ds(o, L)]
            part = (ids16 & 1) * D
            lw = lax.shift_right_logical(cs16, 1) - k0c
            parw = (cs16 & 1) * D
            parp = (slots & 1) * D
            lp = lax.shift_right_logical(slots, 1)
            wsaddr = ws16 * D

            def f_body(f, carry):
                a_s, a_q = carry
                fv = jnp.full((L,), 0, jnp.int32) + f
                xt = plsc.load_gather(tok_s, [slots, part + fv])
                xw = plsc.load_gather(word_s, [lw, parw + fv])
                xp = plsc.load_gather(pos_s, [lp, parp + fv])
                xs = plsc.load_gather(wst_v, [wsaddr + fv])
                x = (xt + xw) + (xp + xs)
                y_s[f, pl.ds(g * L, L)] = x
                return a_s + x, a_q + x * x

            a_s, a_q = lax.fori_loop(
                0, D, f_body,
                (jnp.zeros((L,), jnp.float32), jnp.zeros((L,), jnp.float32)))
            mean = a_s * (1.0 / D)
            var = a_q * (1.0 / D) - mean * mean
            rstd = _rsqrt(var + EPS)

            def f2_body(f, _):
                fv = jnp.full((L,), 0, jnp.int32) + f
                x = y_s[f, pl.ds(g * L, L)]
                gam = plsc.load_gather(gamma_v, [fv])
                bet = plsc.load_gather(beta_v, [fv])
                y_s[f, pl.ds(g * L, L)] = (x - mean) * (rstd * gam) + bet
                return 0

            lax.fori_loop(0, D, f2_body, 0)
            return 0

        lax.fori_loop(0, CHUNK // L, group_body, 0)

        if c + 3 < NCHUNK:
            tok_pend[c + 3] = issue_tok(c + 3)
        if c + 2 < NCHUNK:
            word_pend[c + 2] = issue_word(c + 2)
            pos_pend[c + 2] = issue_pos(c + 2)

        oc = pl.multiple_of(base + c * CHUNK, CHUNK)
        out_pend[s2] = pltpu.async_copy(
            y_s, out_hbm.at[b, :, pl.ds(oc, CHUNK)], osem[s2])

    for d in out_pend:
        if d is not None:
            d.wait()


def kernel(input_ids, word_start, token_table, pos_table, ws_table,
           word_table, gamma, beta):
    ids_flat = input_ids.astype(jnp.int32).reshape(-1)
    ws_flat = word_start.astype(jnp.int32).reshape(-1)
    tok2 = token_table.reshape(token_table.shape[0] // 2, 128)
    pos2 = pos_table.reshape(pos_table.shape[0] // 2, 128)
    word2 = word_table.reshape(word_table.shape[0] // 2, 128)
    wst2 = ws_table.reshape(-1)
    out = _emb_ln(ids_flat, ws_flat, tok2, pos2, wst2, word2, gamma, beta)
    return jnp.swapaxes(out, 1, 2)
